# Initial kernel scaffold; baseline (speedup 1.0000x reference)
import jax
import jax.numpy as jnp
from jax.experimental import pallas as pl


def kernel(x, edge_index, W, b, u0):
    raise NotImplementedError("write your pallas kernel here")


# speedup vs baseline: 28.1110x; 28.1110x over previous
import jax
import jax.numpy as jnp
from jax import lax
from jax.experimental import pallas as pl
from jax.experimental.pallas import tpu as pltpu
from jax.experimental.pallas import tpu_sc as plsc

N = 10000
E = 320000
D = 128
NC = 2
NS = 16
NW = NC * NS
CHUNK = 128
K = -(-E // (NW * CHUNK))
E_PAD = K * NW * CHUNK
NPAD_E = E_PAD - E
N_PAD = 10112
RPS = N_PAD // NS

_MESH = plsc.VectorSubcoreMesh(
    core_axis_name="c", subcore_axis_name="s", num_cores=NC, num_subcores=NS)


def _worker_id():
    return lax.axis_index("c") * NS + lax.axis_index("s")


def _deg_body(dst4, zeros1, deg2, deg_sh, dstv, onesv, stagev):
    c = lax.axis_index("c")
    s = lax.axis_index("s")
    r0 = s * RPS
    pltpu.sync_copy(zeros1.at[pl.ds(r0, RPS)], stagev)
    pltpu.sync_copy(stagev, deg_sh.at[pl.ds(r0, RPS)])
    pltpu.sync_copy(dst4.at[_worker_id()], dstv)
    for i in range(CHUNK // 16):
        onesv[pl.ds(i * 16, 16)] = jnp.full((16,), 1.0, jnp.float32)
    plsc.subcore_barrier()

    def step(j, carry):
        pltpu.sync_copy(onesv, deg_sh.at[dstv.at[j]], add=True)
        return carry

    lax.fori_loop(0, K, step, 0)
    plsc.subcore_barrier()
    pltpu.sync_copy(deg_sh.at[pl.ds(r0, RPS)], stagev)
    pltpu.sync_copy(stagev, deg2.at[pl.ds(c * N_PAD + r0, RPS)])


_deg_call = pl.kernel(
    _deg_body,
    out_type=jax.ShapeDtypeStruct((NC * N_PAD,), jnp.float32),
    mesh=_MESH,
    scratch_types=[
        pltpu.VMEM_SHARED((N_PAD,), jnp.float32),
        pltpu.VMEM((K, CHUNK), jnp.int32),
        pltpu.VMEM((CHUNK,), jnp.float32),
        pltpu.VMEM((RPS,), jnp.float32),
    ],
)


def _mm_body(x_ref, w_ref, u_ref, deg_ref, o_ref):
    W = w_ref[...]
    u = u_ref[...]
    v = u
    for _ in range(3):
        v = lax.dot_general(W, u, (((0,), (0,)), ((), ())))
        v = v / (jnp.sqrt(jnp.sum(v * v)) + 1e-12)
        u = jnp.dot(W, v)
        u = u / (jnp.sqrt(jnp.sum(u * u)) + 1e-12)
    sigma = jnp.sum(u * jnp.dot(W, v))
    w_sn = W / sigma
    h = jnp.dot(x_ref[...], w_sn, preferred_element_type=jnp.float32)
    d = deg_ref[:, 0] + deg_ref[:, 1] + 1.0
    o_ref[...] = h * lax.rsqrt(d)[:, None]


def _mm_call(x_pad, W, u0, deg2):
    return pl.pallas_call(
        _mm_body,
        grid=(NS,),
        in_specs=[
            pl.BlockSpec((RPS, D), lambda i: (i, 0)),
            pl.BlockSpec((D, D), lambda i: (0, 0)),
            pl.BlockSpec((D, 1), lambda i: (0, 0)),
            pl.BlockSpec((RPS, NC), lambda i: (i, 0)),
        ],
        out_specs=pl.BlockSpec((RPS, D), lambda i: (i, 0)),
        out_shape=jax.ShapeDtypeStruct((N_PAD, D), jnp.float32),
    )(x_pad, W, u0, deg2)


def _scat_body(g, src4, dst4, zeros2, s2, acc_sh, srcv, dstv, rows, sem):
    c = lax.axis_index("c")
    s = lax.axis_index("s")
    wid = c * NS + s
    r0 = s * RPS
    pltpu.sync_copy(src4.at[wid], srcv)
    pltpu.sync_copy(dst4.at[wid], dstv)
    pltpu.sync_copy(zeros2, rows)
    for t in range(5):
        n = 128 if t < 4 else RPS - 4 * 128
        pltpu.sync_copy(rows.at[pl.ds(0, n)], acc_sh.at[pl.ds(r0 + t * 128, n)])
    plsc.subcore_barrier()

    def step(j, carry):
        pltpu.async_copy(g.at[srcv.at[j]], rows, sem).wait()
        pltpu.sync_copy(rows, acc_sh.at[dstv.at[j]], add=True)
        return carry

    lax.fori_loop(0, K, step, 0)
    plsc.subcore_barrier()
    for t in range(5):
        n = 128 if t < 4 else RPS - 4 * 128
        pltpu.sync_copy(acc_sh.at[pl.ds(r0 + t * 128, n)], rows.at[pl.ds(0, n)])
        pltpu.sync_copy(rows.at[pl.ds(0, n)], s2.at[c, pl.ds(r0 + t * 128, n)])


_scat_call = pl.kernel(
    _scat_body,
    out_type=jax.ShapeDtypeStruct((NC, N_PAD, D), jnp.float32),
    mesh=_MESH,
    scratch_types=[
        pltpu.VMEM_SHARED((N_PAD, D), jnp.float32),
        pltpu.VMEM((K, CHUNK), jnp.int32),
        pltpu.VMEM((K, CHUNK), jnp.int32),
        pltpu.VMEM((CHUNK, D), jnp.float32),
        pltpu.SemaphoreType.DMA,
    ],
)


def _fin_body(s2_ref, g_ref, deg_ref, b_ref, o_ref):
    d = deg_ref[:, 0] + deg_ref[:, 1] + 1.0
    dinv = lax.rsqrt(d)[:, None]
    acc = s2_ref[0] + s2_ref[1] + g_ref[...]
    o_ref[...] = acc * dinv + b_ref[...]


def _fin_call(s2, g, deg2, b):
    blk = 1000
    return pl.pallas_call(
        _fin_body,
        grid=(N // blk,),
        in_specs=[
            pl.BlockSpec((NC, blk, D), lambda i: (0, i, 0)),
            pl.BlockSpec((blk, D), lambda i: (i, 0)),
            pl.BlockSpec((blk, NC), lambda i: (i, 0)),
            pl.BlockSpec((1, D), lambda i: (0, 0)),
        ],
        out_specs=pl.BlockSpec((blk, D), lambda i: (i, 0)),
        out_shape=jax.ShapeDtypeStruct((N, D), jnp.float32),
    )(s2, g, deg2, b)


def kernel(x, edge_index, W, b, u0):
    src = edge_index[0].astype(jnp.int32)
    dst = edge_index[1].astype(jnp.int32)
    pad_i = jnp.arange(NPAD_E, dtype=jnp.int32)
    src_p = jnp.concatenate([src, (pad_i * 37) % N])
    dst_p = jnp.concatenate([dst, N + pad_i % (N_PAD - N)])
    src4 = src_p.reshape(NW, K, CHUNK)
    dst4 = dst_p.reshape(NW, K, CHUNK)

    zeros1 = jnp.zeros((N_PAD,), jnp.float32)
    zeros2 = jnp.zeros((CHUNK, D), jnp.float32)

    deg2 = _deg_call(dst4, zeros1).reshape(NC, N_PAD).T

    x_pad = jnp.concatenate([x, jnp.zeros((N_PAD - N, D), x.dtype)])
    g = _mm_call(x_pad, W, u0.reshape(D, 1), deg2)

    s2 = _scat_call(g, src4, dst4, zeros2)
    return _fin_call(s2, g, deg2, b.reshape(1, D))


# speedup vs baseline: 33.0814x; 1.1768x over previous
import jax
import jax.numpy as jnp
from jax import lax
from jax.experimental import pallas as pl
from jax.experimental.pallas import tpu as pltpu
from jax.experimental.pallas import tpu_sc as plsc

N = 10000
E = 320000
D = 128
NC = 2
NS = 16
NW = NC * NS
CHUNK = 128
K = 80
KP = K // 2
E_PAD = K * NW * CHUNK
NPAD_E = E_PAD - E
N_PAD = 10112
RPS = N_PAD // NS

_MESH = plsc.VectorSubcoreMesh(
    core_axis_name="c", subcore_axis_name="s", num_cores=NC, num_subcores=NS)


def _worker_id():
    return lax.axis_index("c") * NS + lax.axis_index("s")


def _deg_body(dst4, zeros1, deg2, deg_sh, dstv, onesv, stagev):
    c = lax.axis_index("c")
    s = lax.axis_index("s")
    r0 = s * RPS
    pltpu.sync_copy(zeros1.at[pl.ds(r0, RPS)], stagev)
    pltpu.sync_copy(stagev, deg_sh.at[pl.ds(r0, RPS)])
    pltpu.sync_copy(dst4.at[_worker_id()], dstv)
    for i in range(CHUNK // 16):
        onesv[pl.ds(i * 16, 16)] = jnp.full((16,), 1.0, jnp.float32)
    plsc.subcore_barrier()

    def step(j, carry):
        pltpu.sync_copy(onesv, deg_sh.at[dstv.at[j]], add=True)
        return carry

    lax.fori_loop(0, K, step, 0)
    plsc.subcore_barrier()
    pltpu.sync_copy(deg_sh.at[pl.ds(r0, RPS)], stagev)
    pltpu.sync_copy(stagev, deg2.at[pl.ds(c * N_PAD + r0, RPS)])


_deg_call = pl.kernel(
    _deg_body,
    out_type=jax.ShapeDtypeStruct((NC * N_PAD,), jnp.float32),
    mesh=_MESH,
    scratch_types=[
        pltpu.VMEM_SHARED((N_PAD,), jnp.float32),
        pltpu.VMEM((K, CHUNK), jnp.int32),
        pltpu.VMEM((CHUNK,), jnp.float32),
        pltpu.VMEM((RPS,), jnp.float32),
    ],
)


def _mm_body(x_ref, w_ref, u_ref, deg_ref, o_ref):
    W = w_ref[...]
    u = u_ref[...]
    v = u
    for _ in range(3):
        v = lax.dot_general(W, u, (((0,), (0,)), ((), ())))
        v = v / (jnp.sqrt(jnp.sum(v * v)) + 1e-12)
        u = jnp.dot(W, v)
        u = u / (jnp.sqrt(jnp.sum(u * u)) + 1e-12)
    sigma = jnp.sum(u * jnp.dot(W, v))
    w_sn = W / sigma
    h = jnp.dot(x_ref[...], w_sn, preferred_element_type=jnp.float32)
    d = deg_ref[:, 0] + deg_ref[:, 1] + 1.0
    o_ref[...] = h * lax.rsqrt(d)[:, None]


def _mm_call(x_pad, W, u0, deg2):
    return pl.pallas_call(
        _mm_body,
        grid=(NS,),
        in_specs=[
            pl.BlockSpec((RPS, D), lambda i: (i, 0)),
            pl.BlockSpec((D, D), lambda i: (0, 0)),
            pl.BlockSpec((D, 1), lambda i: (0, 0)),
            pl.BlockSpec((RPS, NC), lambda i: (i, 0)),
        ],
        out_specs=pl.BlockSpec((RPS, D), lambda i: (i, 0)),
        out_shape=jax.ShapeDtypeStruct((N_PAD, D), jnp.float32),
    )(x_pad, W, u0, deg2)


def _scat_body(g, src_ph, dst_ph, zeros2, s2, acc_sh, srcv, dstv, rows, gsem):
    c = lax.axis_index("c")
    s = lax.axis_index("s")
    wid = c * NS + s
    r0 = s * RPS
    pltpu.sync_copy(zeros2, rows.at[0])
    for t in range(5):
        n = 128 if t < 4 else RPS - 4 * 128
        pltpu.sync_copy(rows.at[0, pl.ds(0, n)], acc_sh.at[pl.ds(r0 + t * 128, n)])
    plsc.subcore_barrier()

    for ph in range(2):
        pltpu.sync_copy(src_ph.at[wid * 2 + ph], srcv)
        pltpu.sync_copy(dst_ph.at[wid * 2 + ph], dstv)
        pltpu.async_copy(g.at[srcv.at[0]], rows.at[0], gsem)

        def step(j, carry):
            p = lax.rem(j, 2)
            pltpu.make_async_copy(g.at[srcv.at[j]], rows.at[p], gsem).wait()
            pltpu.async_copy(g.at[srcv.at[j + 1]], rows.at[1 - p], gsem)
            pltpu.sync_copy(rows.at[p], acc_sh.at[dstv.at[j]], add=True)
            return carry

        lax.fori_loop(0, KP, step, 0)
        pltpu.make_async_copy(g.at[srcv.at[KP]], rows.at[lax.rem(KP, 2)], gsem).wait()
    plsc.subcore_barrier()
    for t in range(5):
        n = 128 if t < 4 else RPS - 4 * 128
        pltpu.sync_copy(acc_sh.at[pl.ds(r0 + t * 128, n)], rows.at[0, pl.ds(0, n)])
        pltpu.sync_copy(rows.at[0, pl.ds(0, n)], s2.at[c, pl.ds(r0 + t * 128, n)])


_scat_call = pl.kernel(
    _scat_body,
    out_type=jax.ShapeDtypeStruct((NC, N_PAD, D), jnp.float32),
    mesh=_MESH,
    scratch_types=[
        pltpu.VMEM_SHARED((N_PAD, D), jnp.float32),
        pltpu.VMEM((KP + 1, CHUNK), jnp.int32),
        pltpu.VMEM((KP, CHUNK), jnp.int32),
        pltpu.VMEM((2, CHUNK, D), jnp.float32),
        pltpu.SemaphoreType.DMA,
    ],
)


def _phase_slices(a4):
    n = a4.shape[1] - K + KP
    return jnp.stack([a4[:, 0:n], a4[:, KP:KP + n]], axis=1).reshape(
        NW * 2, n, CHUNK)


def _fin_body(s2_ref, g_ref, deg_ref, b_ref, o_ref):
    d = deg_ref[:, 0] + deg_ref[:, 1] + 1.0
    dinv = lax.rsqrt(d)[:, None]
    acc = s2_ref[0] + s2_ref[1] + g_ref[...]
    o_ref[...] = acc * dinv + b_ref[...]


def _fin_call(s2, g, deg2, b):
    blk = 1000
    return pl.pallas_call(
        _fin_body,
        grid=(N // blk,),
        in_specs=[
            pl.BlockSpec((NC, blk, D), lambda i: (0, i, 0)),
            pl.BlockSpec((blk, D), lambda i: (i, 0)),
            pl.BlockSpec((blk, NC), lambda i: (i, 0)),
            pl.BlockSpec((1, D), lambda i: (0, 0)),
        ],
        out_specs=pl.BlockSpec((blk, D), lambda i: (i, 0)),
        out_shape=jax.ShapeDtypeStruct((N, D), jnp.float32),
    )(s2, g, deg2, b)


def kernel(x, edge_index, W, b, u0):
    src = edge_index[0].astype(jnp.int32)
    dst = edge_index[1].astype(jnp.int32)
    pad_i = jnp.arange(NPAD_E, dtype=jnp.int32)
    src_p = jnp.concatenate([src, (pad_i * 37) % N])
    dst_p = jnp.concatenate([dst, N + pad_i % (N_PAD - N)])
    src4 = src_p.reshape(NW, K, CHUNK)
    dst4 = dst_p.reshape(NW, K, CHUNK)
    src5 = jnp.concatenate([src4, src4[:, :1]], axis=1)
    src_ph = _phase_slices(src5)
    dst_ph = _phase_slices(dst4)

    zeros1 = jnp.zeros((N_PAD,), jnp.float32)
    zeros2 = jnp.zeros((CHUNK, D), jnp.float32)

    deg2 = _deg_call(dst4, zeros1).reshape(NC, N_PAD).T

    x_pad = jnp.concatenate([x, jnp.zeros((N_PAD - N, D), x.dtype)])
    g = _mm_call(x_pad, W, u0.reshape(D, 1), deg2)

    s2 = _scat_call(g, src_ph, dst_ph, zeros2)
    return _fin_call(s2, g, deg2, b.reshape(1, D))


# speedup vs baseline: 36.7603x; 1.1112x over previous
import jax
import jax.numpy as jnp
from jax import lax
from jax.experimental import pallas as pl
from jax.experimental.pallas import tpu as pltpu
from jax.experimental.pallas import tpu_sc as plsc

N = 10000
E = 320000
D = 128
NC = 2
NS = 16
NW = NC * NS
CHUNK = 128
K = 80
KP = K // 2
E_PAD = K * NW * CHUNK
NPAD_E = E_PAD - E
N_PAD = 10112
RPS = N_PAD // NS

_MESH = plsc.VectorSubcoreMesh(
    core_axis_name="c", subcore_axis_name="s", num_cores=NC, num_subcores=NS)


def _worker_id():
    return lax.axis_index("c") * NS + lax.axis_index("s")


def _deg_body(dst4, zeros1, deg2, deg_sh, dstv, onesv, stagev):
    c = lax.axis_index("c")
    s = lax.axis_index("s")
    r0 = s * RPS
    pltpu.sync_copy(zeros1.at[pl.ds(r0, RPS)], stagev)
    pltpu.sync_copy(stagev, deg_sh.at[pl.ds(r0, RPS)])
    pltpu.sync_copy(dst4.at[_worker_id()], dstv)
    for i in range(CHUNK // 16):
        onesv[pl.ds(i * 16, 16)] = jnp.full((16,), 1.0, jnp.float32)
    plsc.subcore_barrier()

    def step(j, carry):
        pltpu.sync_copy(onesv, deg_sh.at[dstv.at[j]], add=True)
        return carry

    lax.fori_loop(0, K, step, 0)
    plsc.subcore_barrier()
    pltpu.sync_copy(deg_sh.at[pl.ds(r0, RPS)], stagev)
    pltpu.sync_copy(stagev, deg2.at[pl.ds(c * N_PAD + r0, RPS)])


_deg_call = pl.kernel(
    _deg_body,
    out_type=jax.ShapeDtypeStruct((NC * N_PAD,), jnp.float32),
    mesh=_MESH,
    scratch_types=[
        pltpu.VMEM_SHARED((N_PAD,), jnp.float32),
        pltpu.VMEM((K, CHUNK), jnp.int32),
        pltpu.VMEM((CHUNK,), jnp.float32),
        pltpu.VMEM((RPS,), jnp.float32),
    ],
)


def _mm_body(x_ref, w_ref, u_ref, deg_ref, o_ref, wsn_ref):
    @pl.when(pl.program_id(0) == 0)
    def _():
        W = w_ref[...]
        u = u_ref[...]
        v = u
        for _ in range(3):
            v = lax.dot_general(W, u, (((0,), (0,)), ((), ())))
            v = v / (jnp.sqrt(jnp.sum(v * v)) + 1e-12)
            u = jnp.dot(W, v)
            u = u / (jnp.sqrt(jnp.sum(u * u)) + 1e-12)
        sigma = jnp.sum(u * jnp.dot(W, v))
        wsn_ref[...] = w_ref[...] / sigma

    h = jnp.dot(x_ref[...], wsn_ref[...], preferred_element_type=jnp.float32)
    d = deg_ref[:, 0] + deg_ref[:, 1] + 1.0
    o_ref[...] = h * lax.rsqrt(d)[:, None]


def _mm_call(x, W, u0, deg2):
    return pl.pallas_call(
        _mm_body,
        grid=(NS,),
        in_specs=[
            pl.BlockSpec((RPS, D), lambda i: (i, 0)),
            pl.BlockSpec((D, D), lambda i: (0, 0)),
            pl.BlockSpec((D, 1), lambda i: (0, 0)),
            pl.BlockSpec((RPS, NC), lambda i: (i, 0)),
        ],
        out_specs=pl.BlockSpec((RPS, D), lambda i: (i, 0)),
        out_shape=jax.ShapeDtypeStruct((N_PAD, D), jnp.float32),
        scratch_shapes=[pltpu.VMEM((D, D), jnp.float32)],
    )(x, W, u0, deg2)


def _scat_body(g, src_ph, dst_ph, zeros2, s2, acc_sh, srcv, dstv, rows, gsem):
    c = lax.axis_index("c")
    s = lax.axis_index("s")
    wid = c * NS + s
    r0 = s * RPS
    pltpu.sync_copy(zeros2, rows.at[0])
    for t in range(5):
        n = 128 if t < 4 else RPS - 4 * 128
        pltpu.sync_copy(rows.at[0, pl.ds(0, n)], acc_sh.at[pl.ds(r0 + t * 128, n)])
    plsc.subcore_barrier()

    for ph in range(2):
        pltpu.sync_copy(src_ph.at[wid * 2 + ph], srcv)
        pltpu.sync_copy(dst_ph.at[wid * 2 + ph], dstv)
        pltpu.async_copy(g.at[srcv.at[0]], rows.at[0], gsem)

        def step(j, carry):
            p = lax.rem(j, 2)
            pltpu.make_async_copy(g.at[srcv.at[j]], rows.at[p], gsem).wait()
            pltpu.async_copy(g.at[srcv.at[j + 1]], rows.at[1 - p], gsem)
            pltpu.sync_copy(rows.at[p], acc_sh.at[dstv.at[j]], add=True)
            return carry

        lax.fori_loop(0, KP, step, 0)
        pltpu.make_async_copy(g.at[srcv.at[KP]], rows.at[lax.rem(KP, 2)], gsem).wait()
    plsc.subcore_barrier()
    for t in range(5):
        n = 128 if t < 4 else RPS - 4 * 128
        pltpu.sync_copy(acc_sh.at[pl.ds(r0 + t * 128, n)], rows.at[0, pl.ds(0, n)])
        pltpu.sync_copy(rows.at[0, pl.ds(0, n)], s2.at[c, pl.ds(r0 + t * 128, n)])


_scat_call = pl.kernel(
    _scat_body,
    out_type=jax.ShapeDtypeStruct((NC, N_PAD, D), jnp.float32),
    mesh=_MESH,
    scratch_types=[
        pltpu.VMEM_SHARED((N_PAD, D), jnp.float32),
        pltpu.VMEM((KP + 1, CHUNK), jnp.int32),
        pltpu.VMEM((KP, CHUNK), jnp.int32),
        pltpu.VMEM((2, CHUNK, D), jnp.float32),
        pltpu.SemaphoreType.DMA,
    ],
)


def _phase_slices(a4):
    n = a4.shape[1] - K + KP
    return jnp.stack([a4[:, 0:n], a4[:, KP:KP + n]], axis=1).reshape(
        NW * 2, n, CHUNK)


def _fin_body(s2_ref, g_ref, deg_ref, b_ref, o_ref):
    d = deg_ref[:, 0] + deg_ref[:, 1] + 1.0
    dinv = lax.rsqrt(d)[:, None]
    acc = s2_ref[0] + s2_ref[1] + g_ref[...]
    o_ref[...] = acc * dinv + b_ref[...]


def _fin_call(s2, g, deg2, b):
    blk = 1000
    return pl.pallas_call(
        _fin_body,
        grid=(N // blk,),
        in_specs=[
            pl.BlockSpec((NC, blk, D), lambda i: (0, i, 0)),
            pl.BlockSpec((blk, D), lambda i: (i, 0)),
            pl.BlockSpec((blk, NC), lambda i: (i, 0)),
            pl.BlockSpec((1, D), lambda i: (0, 0)),
        ],
        out_specs=pl.BlockSpec((blk, D), lambda i: (i, 0)),
        out_shape=jax.ShapeDtypeStruct((N, D), jnp.float32),
    )(s2, g, deg2, b)


def kernel(x, edge_index, W, b, u0):
    src = edge_index[0].astype(jnp.int32)
    dst = edge_index[1].astype(jnp.int32)
    pad_i = jnp.arange(NPAD_E, dtype=jnp.int32)
    src_p = jnp.concatenate([src, (pad_i * 37) % N])
    dst_p = jnp.concatenate([dst, N + pad_i % (N_PAD - N)])
    src4 = src_p.reshape(NW, K, CHUNK)
    dst4 = dst_p.reshape(NW, K, CHUNK)
    src5 = jnp.concatenate([src4, src4[:, :1]], axis=1)
    src_ph = _phase_slices(src5)
    dst_ph = _phase_slices(dst4)

    zeros1 = jnp.zeros((N_PAD,), jnp.float32)
    zeros2 = jnp.zeros((CHUNK, D), jnp.float32)

    deg2 = _deg_call(dst4, zeros1).reshape(NC, N_PAD).T

    g = _mm_call(x, W, u0.reshape(D, 1), deg2)

    s2 = _scat_call(g, src_ph, dst_ph, zeros2)
    return _fin_call(s2, g, deg2, b.reshape(1, D))


# speedup vs baseline: 37.8366x; 1.0293x over previous
import jax
import jax.numpy as jnp
from jax import lax
from jax.experimental import pallas as pl
from jax.experimental.pallas import tpu as pltpu
from jax.experimental.pallas import tpu_sc as plsc

N = 10000
E = 320000
D = 128
NC = 2
NS = 16
NW = NC * NS
CHUNK = 128
K = 80
KP = K // 2
E_PAD = K * NW * CHUNK
NPAD_E = E_PAD - E
N_PAD = 10112
RPS = N_PAD // NS

_MESH = plsc.VectorSubcoreMesh(
    core_axis_name="c", subcore_axis_name="s", num_cores=NC, num_subcores=NS)


def _worker_id():
    return lax.axis_index("c") * NS + lax.axis_index("s")


def _deg_body(dst4, zeros1, deg2, deg_sh, dstv, onesv, stagev, ssem):
    c = lax.axis_index("c")
    s = lax.axis_index("s")
    wid = c * NS + s
    r0 = s * RPS
    pltpu.sync_copy(zeros1.at[pl.ds(r0, RPS)], stagev)
    pltpu.sync_copy(stagev, deg_sh.at[pl.ds(r0, RPS)])
    pltpu.sync_copy(dst4.at[wid], dstv)
    for i in range(CHUNK // 16):
        onesv[pl.ds(i * 16, 16)] = jnp.full((16,), 1.0, jnp.float32)
    plsc.subcore_barrier()

    def step(j, carry):
        pltpu.async_copy(onesv, deg_sh.at[dstv.at[j]], ssem, add=True)
        return carry

    lax.fori_loop(0, K, step, 0)
    pltpu.make_async_copy(dst4.at[wid], dstv, ssem).wait()
    plsc.subcore_barrier()
    pltpu.sync_copy(deg_sh.at[pl.ds(r0, RPS)], stagev)
    pltpu.sync_copy(stagev, deg2.at[pl.ds(c * N_PAD + r0, RPS)])


_deg_call = pl.kernel(
    _deg_body,
    out_type=jax.ShapeDtypeStruct((NC * N_PAD,), jnp.float32),
    mesh=_MESH,
    scratch_types=[
        pltpu.VMEM_SHARED((N_PAD,), jnp.float32),
        pltpu.VMEM((K, CHUNK), jnp.int32),
        pltpu.VMEM((CHUNK,), jnp.float32),
        pltpu.VMEM((RPS,), jnp.float32),
        pltpu.SemaphoreType.DMA,
    ],
)


def _mm_body(x_ref, w_ref, u_ref, deg_ref, o_ref, wsn_ref):
    @pl.when(pl.program_id(0) == 0)
    def _():
        W = w_ref[...]
        u = u_ref[...]
        v = u
        for _ in range(3):
            v = lax.dot_general(W, u, (((0,), (0,)), ((), ())))
            v = v / (jnp.sqrt(jnp.sum(v * v)) + 1e-12)
            u = jnp.dot(W, v)
            u = u / (jnp.sqrt(jnp.sum(u * u)) + 1e-12)
        sigma = jnp.sum(u * jnp.dot(W, v))
        wsn_ref[...] = w_ref[...] / sigma

    h = jnp.dot(x_ref[...], wsn_ref[...], preferred_element_type=jnp.float32)
    d = deg_ref[:, 0] + deg_ref[:, 1] + 1.0
    o_ref[...] = h * lax.rsqrt(d)[:, None]


def _mm_call(x, W, u0, deg2):
    return pl.pallas_call(
        _mm_body,
        grid=(NS,),
        in_specs=[
            pl.BlockSpec((RPS, D), lambda i: (i, 0)),
            pl.BlockSpec((D, D), lambda i: (0, 0)),
            pl.BlockSpec((D, 1), lambda i: (0, 0)),
            pl.BlockSpec((RPS, NC), lambda i: (i, 0)),
        ],
        out_specs=pl.BlockSpec((RPS, D), lambda i: (i, 0)),
        out_shape=jax.ShapeDtypeStruct((N_PAD, D), jnp.float32),
        scratch_shapes=[pltpu.VMEM((D, D), jnp.float32)],
    )(x, W, u0, deg2)


def _scat_body(g, src_ph, dst_ph, zeros2, s2, acc_sh, srcv, dstv, rows, gsem, wsem):
    c = lax.axis_index("c")
    s = lax.axis_index("s")
    wid = c * NS + s
    r0 = s * RPS
    pltpu.sync_copy(zeros2, rows.at[0])
    for t in range(5):
        n = 128 if t < 4 else RPS - 4 * 128
        pltpu.async_copy(
            rows.at[0, pl.ds(0, n)], acc_sh.at[pl.ds(r0 + t * 128, n)], wsem)
    for t in range(5):
        n = 128 if t < 4 else RPS - 4 * 128
        pltpu.make_async_copy(
            rows.at[0, pl.ds(0, n)], acc_sh.at[pl.ds(r0 + t * 128, n)], wsem).wait()
    plsc.subcore_barrier()

    for ph in range(2):
        pltpu.sync_copy(src_ph.at[wid * 2 + ph], srcv)
        pltpu.sync_copy(dst_ph.at[wid * 2 + ph], dstv)
        pltpu.async_copy(g.at[srcv.at[0]], rows.at[0], gsem)

        def step(j, carry):
            p = lax.rem(j, 2)
            pltpu.make_async_copy(g.at[srcv.at[j]], rows.at[p], gsem).wait()
            pltpu.async_copy(g.at[srcv.at[j + 1]], rows.at[1 - p], gsem)
            pltpu.sync_copy(rows.at[p], acc_sh.at[dstv.at[j]], add=True)
            return carry

        lax.fori_loop(0, KP, step, 0)
        pltpu.make_async_copy(g.at[srcv.at[KP]], rows.at[lax.rem(KP, 2)], gsem).wait()
    plsc.subcore_barrier()
    sizes = (128, 128, 128, 128, RPS - 4 * 128)

    def _rd(t):
        return (acc_sh.at[pl.ds(r0 + t * 128, sizes[t])],
                rows.at[t % 2, pl.ds(0, sizes[t])])

    def _wr(t):
        return (rows.at[t % 2, pl.ds(0, sizes[t])],
                s2.at[c, pl.ds(r0 + t * 128, sizes[t])])

    pltpu.async_copy(*_rd(0), gsem)
    for t in range(5):
        pltpu.make_async_copy(*_rd(t), gsem).wait()
        pltpu.async_copy(*_wr(t), wsem)
        if t + 1 < 5:
            if t >= 1:
                pltpu.make_async_copy(*_wr(t - 1), wsem).wait()
            pltpu.async_copy(*_rd(t + 1), gsem)
    pltpu.make_async_copy(*_wr(3), wsem).wait()
    pltpu.make_async_copy(*_wr(4), wsem).wait()


_scat_call = pl.kernel(
    _scat_body,
    out_type=jax.ShapeDtypeStruct((NC, N_PAD, D), jnp.float32),
    mesh=_MESH,
    scratch_types=[
        pltpu.VMEM_SHARED((N_PAD, D), jnp.float32),
        pltpu.VMEM((KP + 1, CHUNK), jnp.int32),
        pltpu.VMEM((KP, CHUNK), jnp.int32),
        pltpu.VMEM((2, CHUNK, D), jnp.float32),
        pltpu.SemaphoreType.DMA,
        pltpu.SemaphoreType.DMA,
    ],
)


def _phase_slices(a4):
    n = a4.shape[1] - K + KP
    return jnp.stack([a4[:, 0:n], a4[:, KP:KP + n]], axis=1).reshape(
        NW * 2, n, CHUNK)


def _fin_body(s2_ref, g_ref, deg_ref, b_ref, o_ref):
    d = deg_ref[:, 0] + deg_ref[:, 1] + 1.0
    dinv = lax.rsqrt(d)[:, None]
    acc = s2_ref[0] + s2_ref[1] + g_ref[...]
    o_ref[...] = acc * dinv + b_ref[...]


def _fin_call(s2, g, deg2, b):
    blk = 1000
    return pl.pallas_call(
        _fin_body,
        grid=(N // blk,),
        in_specs=[
            pl.BlockSpec((NC, blk, D), lambda i: (0, i, 0)),
            pl.BlockSpec((blk, D), lambda i: (i, 0)),
            pl.BlockSpec((blk, NC), lambda i: (i, 0)),
            pl.BlockSpec((1, D), lambda i: (0, 0)),
        ],
        out_specs=pl.BlockSpec((blk, D), lambda i: (i, 0)),
        out_shape=jax.ShapeDtypeStruct((N, D), jnp.float32),
    )(s2, g, deg2, b)


def kernel(x, edge_index, W, b, u0):
    src = edge_index[0].astype(jnp.int32)
    dst = edge_index[1].astype(jnp.int32)
    pad_i = jnp.arange(NPAD_E, dtype=jnp.int32)
    src_p = jnp.concatenate([src, (pad_i * 37) % N])
    dst_p = jnp.concatenate([dst, N + pad_i % (N_PAD - N)])
    src4 = src_p.reshape(NW, K, CHUNK)
    dst4 = dst_p.reshape(NW, K, CHUNK)
    src5 = jnp.concatenate([src4, src4[:, :1]], axis=1)
    src_ph = _phase_slices(src5)
    dst_ph = _phase_slices(dst4)

    zeros1 = jnp.zeros((N_PAD,), jnp.float32)
    zeros2 = jnp.zeros((CHUNK, D), jnp.float32)

    deg2 = _deg_call(dst4, zeros1).reshape(NC, N_PAD).T

    g = _mm_call(x, W, u0.reshape(D, 1), deg2)

    s2 = _scat_call(g, src_ph, dst_ph, zeros2)
    return _fin_call(s2, g, deg2, b.reshape(1, D))


# speedup vs baseline: 37.9123x; 1.0020x over previous
import jax
import jax.numpy as jnp
from jax import lax
from jax.experimental import pallas as pl
from jax.experimental.pallas import tpu as pltpu
from jax.experimental.pallas import tpu_sc as plsc

N = 10000
E = 320000
D = 128
NC = 2
NS = 16
NW = NC * NS
CHUNK = 128
K = 80
KP = K // 2
E_PAD = K * NW * CHUNK
NPAD_E = E_PAD - E
N_PAD = 10112
RPS = N_PAD // NS

_MESH = plsc.VectorSubcoreMesh(
    core_axis_name="c", subcore_axis_name="s", num_cores=NC, num_subcores=NS)


def _worker_id():
    return lax.axis_index("c") * NS + lax.axis_index("s")


def _deg_body(dst4, zeros1, deg2, deg_sh, dstv, onesv, stagev, ssem):
    c = lax.axis_index("c")
    s = lax.axis_index("s")
    wid = c * NS + s
    r0 = s * RPS
    pltpu.sync_copy(zeros1.at[pl.ds(r0, RPS)], stagev)
    pltpu.sync_copy(stagev, deg_sh.at[pl.ds(r0, RPS)])
    pltpu.sync_copy(dst4.at[wid], dstv)
    for i in range(CHUNK // 16):
        onesv[pl.ds(i * 16, 16)] = jnp.full((16,), 1.0, jnp.float32)
    plsc.subcore_barrier()

    def step(j, carry):
        pltpu.async_copy(onesv, deg_sh.at[dstv.at[j]], ssem, add=True)
        return carry

    lax.fori_loop(0, K, step, 0)
    pltpu.make_async_copy(dst4.at[wid], dstv, ssem).wait()
    plsc.subcore_barrier()
    pltpu.sync_copy(deg_sh.at[pl.ds(r0, RPS)], stagev)
    pltpu.sync_copy(stagev, deg2.at[pl.ds(c * N_PAD + r0, RPS)])


_deg_call = pl.kernel(
    _deg_body,
    out_type=jax.ShapeDtypeStruct((NC * N_PAD,), jnp.float32),
    mesh=_MESH,
    scratch_types=[
        pltpu.VMEM_SHARED((N_PAD,), jnp.float32),
        pltpu.VMEM((K, CHUNK), jnp.int32),
        pltpu.VMEM((CHUNK,), jnp.float32),
        pltpu.VMEM((RPS,), jnp.float32),
        pltpu.SemaphoreType.DMA,
    ],
)


def _mm_body(x_ref, w_ref, u_ref, deg_ref, o_ref, wsn_ref):
    @pl.when(pl.program_id(0) == 0)
    def _():
        W = w_ref[...]
        u = u_ref[...]
        v = u
        for _ in range(3):
            v = lax.dot_general(W, u, (((0,), (0,)), ((), ())))
            v = v / (jnp.sqrt(jnp.sum(v * v)) + 1e-12)
            u = jnp.dot(W, v)
            u = u / (jnp.sqrt(jnp.sum(u * u)) + 1e-12)
        sigma = jnp.sum(u * jnp.dot(W, v))
        wsn_ref[...] = w_ref[...] / sigma

    h = jnp.dot(x_ref[...], wsn_ref[...], preferred_element_type=jnp.float32)
    d = deg_ref[:, 0] + deg_ref[:, 1] + 1.0
    o_ref[...] = h * lax.rsqrt(d)[:, None]


def _mm_call(x, W, u0, deg2):
    return pl.pallas_call(
        _mm_body,
        grid=(NS,),
        in_specs=[
            pl.BlockSpec((RPS, D), lambda i: (i, 0)),
            pl.BlockSpec((D, D), lambda i: (0, 0)),
            pl.BlockSpec((D, 1), lambda i: (0, 0)),
            pl.BlockSpec((RPS, NC), lambda i: (i, 0)),
        ],
        out_specs=pl.BlockSpec((RPS, D), lambda i: (i, 0)),
        out_shape=jax.ShapeDtypeStruct((N_PAD, D), jnp.float32),
        scratch_shapes=[pltpu.VMEM((D, D), jnp.float32)],
    )(x, W, u0, deg2)


def _scat_body(g, src_ph, dst_ph, zeros2, s2, acc_sh, srcv, dstv, rows, trashv,
               gsem, wsem):
    c = lax.axis_index("c")
    s = lax.axis_index("s")
    wid = c * NS + s
    r0 = s * RPS
    pltpu.sync_copy(zeros2, rows.at[0])
    for t in range(5):
        n = 128 if t < 4 else RPS - 4 * 128
        pltpu.async_copy(
            rows.at[0, pl.ds(0, n)], acc_sh.at[pl.ds(r0 + t * 128, n)], wsem)
    for t in range(5):
        n = 128 if t < 4 else RPS - 4 * 128
        pltpu.make_async_copy(
            rows.at[0, pl.ds(0, n)], acc_sh.at[pl.ds(r0 + t * 128, n)], wsem).wait()
    plsc.subcore_barrier()

    for k in range(CHUNK // 16):
        trashv[pl.ds(k * 16, 16)] = N + k * 12 + lax.iota(jnp.int32, 16)

    for ph in range(2):
        pltpu.sync_copy(src_ph.at[wid * 2 + ph], srcv)
        pltpu.sync_copy(dst_ph.at[wid * 2 + ph], dstv)
        pltpu.async_copy(g.at[srcv.at[0]], rows.at[0], gsem)
        pltpu.async_copy(rows.at[1], acc_sh.at[trashv], wsem, add=True)

        def step(j, carry):
            p = lax.rem(j, 2)
            pltpu.make_async_copy(rows.at[1 - p], acc_sh.at[trashv], wsem).wait()
            pltpu.make_async_copy(g.at[srcv.at[j]], rows.at[p], gsem).wait()
            pltpu.async_copy(g.at[srcv.at[j + 1]], rows.at[1 - p], gsem)
            pltpu.async_copy(rows.at[p], acc_sh.at[dstv.at[j]], wsem, add=True)
            return carry

        lax.fori_loop(0, KP, step, 0)
        pltpu.make_async_copy(rows.at[0], acc_sh.at[trashv], wsem).wait()
        pltpu.make_async_copy(g.at[srcv.at[KP]], rows.at[lax.rem(KP, 2)], gsem).wait()
    plsc.subcore_barrier()
    sizes = (128, 128, 128, 128, RPS - 4 * 128)

    def _rd(t):
        return (acc_sh.at[pl.ds(r0 + t * 128, sizes[t])],
                rows.at[t % 2, pl.ds(0, sizes[t])])

    def _wr(t):
        return (rows.at[t % 2, pl.ds(0, sizes[t])],
                s2.at[c, pl.ds(r0 + t * 128, sizes[t])])

    pltpu.async_copy(*_rd(0), gsem)
    for t in range(5):
        pltpu.make_async_copy(*_rd(t), gsem).wait()
        pltpu.async_copy(*_wr(t), wsem)
        if t + 1 < 5:
            if t >= 1:
                pltpu.make_async_copy(*_wr(t - 1), wsem).wait()
            pltpu.async_copy(*_rd(t + 1), gsem)
    pltpu.make_async_copy(*_wr(3), wsem).wait()
    pltpu.make_async_copy(*_wr(4), wsem).wait()


_scat_call = pl.kernel(
    _scat_body,
    out_type=jax.ShapeDtypeStruct((NC, N_PAD, D), jnp.float32),
    mesh=_MESH,
    scratch_types=[
        pltpu.VMEM_SHARED((N_PAD, D), jnp.float32),
        pltpu.VMEM((KP + 1, CHUNK), jnp.int32),
        pltpu.VMEM((KP, CHUNK), jnp.int32),
        pltpu.VMEM((2, CHUNK, D), jnp.float32),
        pltpu.VMEM((CHUNK,), jnp.int32),
        pltpu.SemaphoreType.DMA,
        pltpu.SemaphoreType.DMA,
    ],
)


def _phase_slices(a4):
    n = a4.shape[1] - K + KP
    return jnp.stack([a4[:, 0:n], a4[:, KP:KP + n]], axis=1).reshape(
        NW * 2, n, CHUNK)


def _fin_body(s2_ref, g_ref, deg_ref, b_ref, o_ref):
    d = deg_ref[:, 0] + deg_ref[:, 1] + 1.0
    dinv = lax.rsqrt(d)[:, None]
    acc = s2_ref[0] + s2_ref[1] + g_ref[...]
    o_ref[...] = acc * dinv + b_ref[...]


def _fin_call(s2, g, deg2, b):
    blk = 1000
    return pl.pallas_call(
        _fin_body,
        grid=(N // blk,),
        in_specs=[
            pl.BlockSpec((NC, blk, D), lambda i: (0, i, 0)),
            pl.BlockSpec((blk, D), lambda i: (i, 0)),
            pl.BlockSpec((blk, NC), lambda i: (i, 0)),
            pl.BlockSpec((1, D), lambda i: (0, 0)),
        ],
        out_specs=pl.BlockSpec((blk, D), lambda i: (i, 0)),
        out_shape=jax.ShapeDtypeStruct((N, D), jnp.float32),
    )(s2, g, deg2, b)


def kernel(x, edge_index, W, b, u0):
    src = edge_index[0].astype(jnp.int32)
    dst = edge_index[1].astype(jnp.int32)
    pad_i = jnp.arange(NPAD_E, dtype=jnp.int32)
    src_p = jnp.concatenate([src, (pad_i * 37) % N])
    dst_p = jnp.concatenate([dst, N + pad_i % (N_PAD - N)])
    src4 = src_p.reshape(NW, K, CHUNK)
    dst4 = dst_p.reshape(NW, K, CHUNK)
    src5 = jnp.concatenate([src4, src4[:, :1]], axis=1)
    src_ph = _phase_slices(src5)
    dst_ph = _phase_slices(dst4)

    zeros1 = jnp.zeros((N_PAD,), jnp.float32)
    zeros2 = jnp.zeros((CHUNK, D), jnp.float32)

    deg2 = _deg_call(dst4, zeros1).reshape(NC, N_PAD).T

    g = _mm_call(x, W, u0.reshape(D, 1), deg2)

    s2 = _scat_call(g, src_ph, dst_ph, zeros2)
    return _fin_call(s2, g, deg2, b.reshape(1, D))


# speedup vs baseline: 39.6469x; 1.0458x over previous
import jax
import jax.numpy as jnp
from jax import lax
from jax.experimental import pallas as pl
from jax.experimental.pallas import tpu as pltpu
from jax.experimental.pallas import tpu_sc as plsc

N = 10000
E = 320000
D = 128
NC = 2
NS = 16
NW = NC * NS
CHUNK = 128
K = 80
KP = K // 2
E_PAD = K * NW * CHUNK
NPAD_E = E_PAD - E
N_PAD = 10112
RPS = N_PAD // NS

_MESH = plsc.VectorSubcoreMesh(
    core_axis_name="c", subcore_axis_name="s", num_cores=NC, num_subcores=NS)


def _worker_id():
    return lax.axis_index("c") * NS + lax.axis_index("s")


def _deg_body(dst4, zeros1, deg2, deg_sh, dstv, onesv, stagev, ssem):
    c = lax.axis_index("c")
    s = lax.axis_index("s")
    wid = c * NS + s
    r0 = s * RPS
    pltpu.sync_copy(zeros1.at[pl.ds(r0, RPS)], stagev)
    pltpu.sync_copy(stagev, deg_sh.at[pl.ds(r0, RPS)])
    pltpu.sync_copy(dst4.at[wid], dstv)
    for i in range(CHUNK // 16):
        onesv[pl.ds(i * 16, 16)] = jnp.full((16,), 1.0, jnp.float32)
    plsc.subcore_barrier()

    def step(j, carry):
        pltpu.async_copy(onesv, deg_sh.at[dstv.at[j]], ssem, add=True)
        return carry

    lax.fori_loop(0, K, step, 0)
    pltpu.make_async_copy(dst4.at[wid], dstv, ssem).wait()
    plsc.subcore_barrier()
    pltpu.sync_copy(deg_sh.at[pl.ds(r0, RPS)], stagev)
    pltpu.sync_copy(stagev, deg2.at[pl.ds(c * N_PAD + r0, RPS)])


_deg_call = pl.kernel(
    _deg_body,
    out_type=jax.ShapeDtypeStruct((NC * N_PAD,), jnp.float32),
    mesh=_MESH,
    scratch_types=[
        pltpu.VMEM_SHARED((N_PAD,), jnp.float32),
        pltpu.VMEM((K, CHUNK), jnp.int32),
        pltpu.VMEM((CHUNK,), jnp.float32),
        pltpu.VMEM((RPS,), jnp.float32),
        pltpu.SemaphoreType.DMA,
    ],
)


def _mm_body(x_ref, w_ref, u_ref, deg_ref, o_ref, wsn_ref):
    @pl.when(pl.program_id(0) == 0)
    def _():
        W = w_ref[...]
        u = u_ref[...]
        v = u
        for _ in range(3):
            v = lax.dot_general(W, u, (((0,), (0,)), ((), ())))
            v = v / (jnp.sqrt(jnp.sum(v * v)) + 1e-12)
            u = jnp.dot(W, v)
            u = u / (jnp.sqrt(jnp.sum(u * u)) + 1e-12)
        sigma = jnp.sum(u * jnp.dot(W, v))
        wsn_ref[...] = w_ref[...] / sigma

    h = jnp.dot(x_ref[...], wsn_ref[...], preferred_element_type=jnp.float32)
    d = deg_ref[:, 0] + deg_ref[:, 1] + 1.0
    o_ref[...] = h * lax.rsqrt(d)[:, None]


def _mm_call(x, W, u0, deg2):
    blk = N_PAD // 4
    return pl.pallas_call(
        _mm_body,
        grid=(4,),
        in_specs=[
            pl.BlockSpec((blk, D), lambda i: (i, 0)),
            pl.BlockSpec((D, D), lambda i: (0, 0)),
            pl.BlockSpec((D, 1), lambda i: (0, 0)),
            pl.BlockSpec((blk, NC), lambda i: (i, 0)),
        ],
        out_specs=pl.BlockSpec((blk, D), lambda i: (i, 0)),
        out_shape=jax.ShapeDtypeStruct((N_PAD, D), jnp.float32),
        scratch_shapes=[pltpu.VMEM((D, D), jnp.float32)],
    )(x, W, u0, deg2)


def _scat_body(g, src_ph, dst_ph, zeros2, s2, acc_sh, srcv, dstv, rows, trashv,
               gsem, wsem):
    c = lax.axis_index("c")
    s = lax.axis_index("s")
    wid = c * NS + s
    r0 = s * RPS
    pltpu.sync_copy(zeros2, rows.at[0])
    for t in range(5):
        n = 128 if t < 4 else RPS - 4 * 128
        pltpu.async_copy(
            rows.at[0, pl.ds(0, n)], acc_sh.at[pl.ds(r0 + t * 128, n)], wsem)
    for t in range(5):
        n = 128 if t < 4 else RPS - 4 * 128
        pltpu.make_async_copy(
            rows.at[0, pl.ds(0, n)], acc_sh.at[pl.ds(r0 + t * 128, n)], wsem).wait()
    plsc.subcore_barrier()

    for k in range(CHUNK // 16):
        trashv[pl.ds(k * 16, 16)] = N + k * 12 + lax.iota(jnp.int32, 16)

    for ph in range(2):
        pltpu.sync_copy(src_ph.at[wid * 2 + ph], srcv)
        pltpu.sync_copy(dst_ph.at[wid * 2 + ph], dstv)
        pltpu.async_copy(g.at[srcv.at[0]], rows.at[0], gsem)
        pltpu.async_copy(rows.at[1], acc_sh.at[trashv], wsem, add=True)

        def step(j, carry):
            p = lax.rem(j, 2)
            pltpu.make_async_copy(rows.at[1 - p], acc_sh.at[trashv], wsem).wait()
            pltpu.make_async_copy(g.at[srcv.at[j]], rows.at[p], gsem).wait()
            pltpu.async_copy(g.at[srcv.at[j + 1]], rows.at[1 - p], gsem)
            pltpu.async_copy(rows.at[p], acc_sh.at[dstv.at[j]], wsem, add=True)
            return carry

        lax.fori_loop(0, KP, step, 0)
        pltpu.make_async_copy(rows.at[0], acc_sh.at[trashv], wsem).wait()
        pltpu.make_async_copy(g.at[srcv.at[KP]], rows.at[lax.rem(KP, 2)], gsem).wait()
    plsc.subcore_barrier()
    sizes = (128, 128, 128, 128, RPS - 4 * 128)

    def _rd(t):
        return (acc_sh.at[pl.ds(r0 + t * 128, sizes[t])],
                rows.at[t % 2, pl.ds(0, sizes[t])])

    def _wr(t):
        return (rows.at[t % 2, pl.ds(0, sizes[t])],
                s2.at[c, pl.ds(r0 + t * 128, sizes[t])])

    pltpu.async_copy(*_rd(0), gsem)
    for t in range(5):
        pltpu.make_async_copy(*_rd(t), gsem).wait()
        pltpu.async_copy(*_wr(t), wsem)
        if t + 1 < 5:
            if t >= 1:
                pltpu.make_async_copy(*_wr(t - 1), wsem).wait()
            pltpu.async_copy(*_rd(t + 1), gsem)
    pltpu.make_async_copy(*_wr(3), wsem).wait()
    pltpu.make_async_copy(*_wr(4), wsem).wait()


_scat_call = pl.kernel(
    _scat_body,
    out_type=jax.ShapeDtypeStruct((NC, N_PAD, D), jnp.float32),
    mesh=_MESH,
    scratch_types=[
        pltpu.VMEM_SHARED((N_PAD, D), jnp.float32),
        pltpu.VMEM((KP + 1, CHUNK), jnp.int32),
        pltpu.VMEM((KP, CHUNK), jnp.int32),
        pltpu.VMEM((2, CHUNK, D), jnp.float32),
        pltpu.VMEM((CHUNK,), jnp.int32),
        pltpu.SemaphoreType.DMA,
        pltpu.SemaphoreType.DMA,
    ],
)


def _phase_slices(a4):
    n = a4.shape[1] - K + KP
    return jnp.stack([a4[:, 0:n], a4[:, KP:KP + n]], axis=1).reshape(
        NW * 2, n, CHUNK)


def _fin_body(s2_ref, g_ref, deg_ref, b_ref, o_ref):
    d = deg_ref[:, 0] + deg_ref[:, 1] + 1.0
    dinv = lax.rsqrt(d)[:, None]
    acc = s2_ref[0] + s2_ref[1] + g_ref[...]
    o_ref[...] = acc * dinv + b_ref[...]


def _fin_call(s2, g, deg2, b):
    blk = 2000
    return pl.pallas_call(
        _fin_body,
        grid=(N // blk,),
        in_specs=[
            pl.BlockSpec((NC, blk, D), lambda i: (0, i, 0)),
            pl.BlockSpec((blk, D), lambda i: (i, 0)),
            pl.BlockSpec((blk, NC), lambda i: (i, 0)),
            pl.BlockSpec((1, D), lambda i: (0, 0)),
        ],
        out_specs=pl.BlockSpec((blk, D), lambda i: (i, 0)),
        out_shape=jax.ShapeDtypeStruct((N, D), jnp.float32),
    )(s2, g, deg2, b)


def kernel(x, edge_index, W, b, u0):
    src = edge_index[0].astype(jnp.int32)
    dst = edge_index[1].astype(jnp.int32)
    pad_i = jnp.arange(NPAD_E, dtype=jnp.int32)
    src_p = jnp.concatenate([src, (pad_i * 37) % N])
    dst_p = jnp.concatenate([dst, N + pad_i % (N_PAD - N)])
    src4 = src_p.reshape(NW, K, CHUNK)
    dst4 = dst_p.reshape(NW, K, CHUNK)
    src5 = jnp.concatenate([src4, src4[:, :1]], axis=1)
    src_ph = _phase_slices(src5)
    dst_ph = _phase_slices(dst4)

    zeros1 = jnp.zeros((N_PAD,), jnp.float32)
    zeros2 = jnp.zeros((CHUNK, D), jnp.float32)

    deg2 = _deg_call(dst4, zeros1).reshape(NC, N_PAD).T

    g = _mm_call(x, W, u0.reshape(D, 1), deg2)

    s2 = _scat_call(g, src_ph, dst_ph, zeros2)
    return _fin_call(s2, g, deg2, b.reshape(1, D))
